# baseline (speedup 1.0000x reference)
import jax
import jax.numpy as jnp
from jax import lax
from jax.experimental import pallas as pl
from jax.experimental.pallas import tpu as pltpu

B = 8
H_ENC = 1024
H_DEC = 1024
TOTAL = 16384
SEG = TOTAL // B

_DN_T = (((1,), (1,)), ((), ()))


def _copy_seg(enc_hbm, buf, sem, seg_idx, slot):
    return pltpu.make_async_copy(
        enc_hbm.at[pl.ds(seg_idx * SEG, SEG), :],
        buf.at[slot],
        sem.at[slot],
    )


def _attn_body(hs_ref, enc_hbm, w_ref, v_ref, out_ref, buf, sem):
    i = pl.program_id(0)
    slot = lax.rem(i, 4)

    @pl.when(i == 0)
    def _prologue():
        _copy_seg(enc_hbm, buf, sem, 0, 0).start()
        _copy_seg(enc_hbm, buf, sem, 1, 1).start()

    @pl.when(i < B - 2)
    def _prefetch():
        _copy_seg(enc_hbm, buf, sem, i + 2, lax.rem(i + 2, 4)).start()

    wd = w_ref[:, :H_DEC]
    we = w_ref[:, H_DEC:]
    hproj_all = lax.dot_general(hs_ref[...], wd, _DN_T,
                                preferred_element_type=jnp.float32)
    mask = (lax.broadcasted_iota(jnp.int32, (B, 1), 0) == i).astype(jnp.float32)
    row = jnp.sum(hproj_all * mask, axis=0, keepdims=True)

    _copy_seg(enc_hbm, buf, sem, i, slot).wait()
    x = lax.dot_general(buf[slot], we, _DN_T,
                        preferred_element_type=jnp.float32)
    energy = jnp.tanh(x + row)
    s = jnp.dot(energy, v_ref[...], preferred_element_type=jnp.float32)
    m = jnp.max(s)
    e = jnp.exp(s - m)
    out_ref[...] = e / jnp.sum(e)


def kernel(hidden_states, encoder_output, tree_sizes, W, v):
    del tree_sizes
    out = pl.pallas_call(
        _attn_body,
        grid=(B,),
        in_specs=[
            pl.BlockSpec((B, H_DEC), lambda i: (0, 0)),
            pl.BlockSpec(memory_space=pl.ANY),
            pl.BlockSpec((H_ENC, H_DEC + H_ENC), lambda i: (0, 0)),
            pl.BlockSpec((H_ENC, 1), lambda i: (0, 0)),
        ],
        out_specs=pl.BlockSpec((SEG, 1), lambda i: (i, 0)),
        out_shape=jax.ShapeDtypeStruct((TOTAL, 1), jnp.float32),
        scratch_shapes=[
            pltpu.VMEM((4, SEG, H_ENC), jnp.float32),
            pltpu.SemaphoreType.DMA((4,)),
        ],
        compiler_params=pltpu.CompilerParams(
            dimension_semantics=("arbitrary",),
            vmem_limit_bytes=100 * 1024 * 1024,
        ),
    )(hidden_states, encoder_output, W, v)
    return out


# speedup vs baseline: 1.0504x; 1.0504x over previous
import jax
import jax.numpy as jnp
from jax import lax
from jax.experimental import pallas as pl
from jax.experimental.pallas import tpu as pltpu

B = 8
H_ENC = 1024
H_DEC = 1024
TOTAL = 16384
SEG = TOTAL // B
SEGS_PER_STEP = 2
BLK = SEG * SEGS_PER_STEP
NSTEP = TOTAL // BLK

_DN_T = (((1,), (1,)), ((), ()))


def _copy_blk(enc_hbm, buf, sem, blk_idx, slot):
    return pltpu.make_async_copy(
        enc_hbm.at[pl.ds(blk_idx * BLK, BLK), :],
        buf.at[slot],
        sem.at[slot],
    )


def _attn_body(hs_ref, enc_hbm, w_ref, v_ref, out_ref, buf, sem):
    i = pl.program_id(0)
    slot = lax.rem(i, 2)

    @pl.when(i == 0)
    def _prologue():
        _copy_blk(enc_hbm, buf, sem, 0, 0).start()

    @pl.when(i < NSTEP - 1)
    def _prefetch():
        _copy_blk(enc_hbm, buf, sem, i + 1, lax.rem(i + 1, 2)).start()

    wd = w_ref[:, :H_DEC]
    we = w_ref[:, H_DEC:]
    hproj_all = lax.dot_general(hs_ref[...], wd, _DN_T,
                                preferred_element_type=jnp.float32)
    iota = lax.broadcasted_iota(jnp.int32, (B, 1), 0)

    _copy_blk(enc_hbm, buf, sem, i, slot).wait()
    x = lax.dot_general(buf[slot], we, _DN_T,
                        preferred_element_type=jnp.float32)
    for k in range(SEGS_PER_STEP):
        mask = (iota == i * SEGS_PER_STEP + k).astype(jnp.float32)
        row = jnp.sum(hproj_all * mask, axis=0, keepdims=True)
        energy = jnp.tanh(x[k * SEG:(k + 1) * SEG, :] + row)
        sk = jnp.dot(energy, v_ref[...],
                     preferred_element_type=jnp.float32)
        m = jnp.max(sk)
        e = jnp.exp(sk - m)
        out_ref[k * SEG:(k + 1) * SEG, :] = e / jnp.sum(e)


def kernel(hidden_states, encoder_output, tree_sizes, W, v):
    del tree_sizes
    out = pl.pallas_call(
        _attn_body,
        grid=(NSTEP,),
        in_specs=[
            pl.BlockSpec((B, H_DEC), lambda i: (0, 0)),
            pl.BlockSpec(memory_space=pl.ANY),
            pl.BlockSpec((H_ENC, H_DEC + H_ENC), lambda i: (0, 0)),
            pl.BlockSpec((H_ENC, 1), lambda i: (0, 0)),
        ],
        out_specs=pl.BlockSpec((BLK, 1), lambda i: (i, 0)),
        out_shape=jax.ShapeDtypeStruct((TOTAL, 1), jnp.float32),
        scratch_shapes=[
            pltpu.VMEM((2, BLK, H_ENC), jnp.float32),
            pltpu.SemaphoreType.DMA((2,)),
        ],
        compiler_params=pltpu.CompilerParams(
            dimension_semantics=("arbitrary",),
            vmem_limit_bytes=100 * 1024 * 1024,
        ),
    )(hidden_states, encoder_output, W, v)
    return out


# speedup vs baseline: 1.2224x; 1.1637x over previous
import jax
import jax.numpy as jnp
from jax import lax
from jax.experimental import pallas as pl
from jax.experimental.pallas import tpu as pltpu

B = 8
H_ENC = 1024
H_DEC = 1024
TOTAL = 16384
SEG = TOTAL // B
SEGS_PER_STEP = 2
BLK = SEG * SEGS_PER_STEP
NSTEP = TOTAL // BLK

_DN_T = (((1,), (1,)), ((), ()))
_DN_VT = (((0,), (1,)), ((), ()))


def _attn_body(hs_ref, enc_ref, w_ref, v_ref, out_ref):
    i = pl.program_id(0)
    wd = w_ref[:, :H_DEC]
    we = w_ref[:, H_DEC:]
    hproj_all = lax.dot_general(hs_ref[...], wd, _DN_T,
                                preferred_element_type=jnp.float32)
    x = lax.dot_general(enc_ref[...], we, _DN_T,
                        preferred_element_type=jnp.float32)
    iota = lax.broadcasted_iota(jnp.int32, (B, 1), 0)
    for k in range(SEGS_PER_STEP):
        mask = (iota == i * SEGS_PER_STEP + k).astype(jnp.float32)
        row = jnp.sum(hproj_all * mask, axis=0, keepdims=True)
        energy = jnp.tanh(x[k * SEG:(k + 1) * SEG, :] + row)
        s_row = lax.dot_general(v_ref[...], energy, _DN_VT,
                                preferred_element_type=jnp.float32)
        m = jnp.max(s_row)
        e = jnp.exp(s_row - m)
        out_ref[:, k, :] = e / jnp.sum(e)


def kernel(hidden_states, encoder_output, tree_sizes, W, v):
    del tree_sizes
    out = pl.pallas_call(
        _attn_body,
        grid=(NSTEP,),
        in_specs=[
            pl.BlockSpec((B, H_DEC), lambda i: (0, 0)),
            pl.BlockSpec((BLK, H_ENC), lambda i: (i, 0)),
            pl.BlockSpec((H_ENC, H_DEC + H_ENC), lambda i: (0, 0)),
            pl.BlockSpec((H_ENC, 1), lambda i: (0, 0)),
        ],
        out_specs=pl.BlockSpec((1, SEGS_PER_STEP, SEG), lambda i: (i, 0, 0)),
        out_shape=jax.ShapeDtypeStruct((NSTEP, SEGS_PER_STEP, SEG), jnp.float32),
        compiler_params=pltpu.CompilerParams(
            dimension_semantics=("parallel",),
            vmem_limit_bytes=100 * 1024 * 1024,
        ),
    )(hidden_states, encoder_output, W, v)
    return out.reshape(TOTAL, 1)


# speedup vs baseline: 1.2231x; 1.0005x over previous
import jax
import jax.numpy as jnp
from jax import lax
from jax.experimental import pallas as pl
from jax.experimental.pallas import tpu as pltpu

B = 8
H_ENC = 1024
H_DEC = 1024
TOTAL = 16384
SEG = TOTAL // B
SEGS_PER_STEP = 1
BLK = SEG * SEGS_PER_STEP
NSTEP = TOTAL // BLK

_DN_T = (((1,), (1,)), ((), ()))
_DN_VT = (((0,), (1,)), ((), ()))


def _attn_body(hs_ref, enc_ref, w_ref, v_ref, out_ref):
    i = pl.program_id(0)
    wd = w_ref[:, :H_DEC]
    we = w_ref[:, H_DEC:]
    hproj_all = lax.dot_general(hs_ref[...], wd, _DN_T,
                                preferred_element_type=jnp.float32)
    x = lax.dot_general(enc_ref[...], we, _DN_T,
                        preferred_element_type=jnp.float32)
    iota = lax.broadcasted_iota(jnp.int32, (B, 1), 0)
    for k in range(SEGS_PER_STEP):
        mask = (iota == i * SEGS_PER_STEP + k).astype(jnp.float32)
        row = jnp.sum(hproj_all * mask, axis=0, keepdims=True)
        energy = jnp.tanh(x[k * SEG:(k + 1) * SEG, :] + row)
        s_row = lax.dot_general(v_ref[...], energy, _DN_VT,
                                preferred_element_type=jnp.float32)
        m = jnp.max(s_row)
        e = jnp.exp(s_row - m)
        out_ref[:, k, :] = e / jnp.sum(e)


def kernel(hidden_states, encoder_output, tree_sizes, W, v):
    del tree_sizes
    out = pl.pallas_call(
        _attn_body,
        grid=(NSTEP,),
        in_specs=[
            pl.BlockSpec((B, H_DEC), lambda i: (0, 0)),
            pl.BlockSpec((BLK, H_ENC), lambda i: (i, 0)),
            pl.BlockSpec((H_ENC, H_DEC + H_ENC), lambda i: (0, 0)),
            pl.BlockSpec((H_ENC, 1), lambda i: (0, 0)),
        ],
        out_specs=pl.BlockSpec((1, SEGS_PER_STEP, SEG), lambda i: (i, 0, 0)),
        out_shape=jax.ShapeDtypeStruct((NSTEP, SEGS_PER_STEP, SEG), jnp.float32),
        compiler_params=pltpu.CompilerParams(
            dimension_semantics=("parallel",),
            vmem_limit_bytes=100 * 1024 * 1024,
        ),
    )(hidden_states, encoder_output, W, v)
    return out.reshape(TOTAL, 1)
